# baseline (device time: 152768 ns/iter reference)
import jax
import jax.numpy as jnp
from jax import lax
from jax.experimental import pallas as pl
from jax.experimental.pallas import tpu as pltpu

N_DEV = 32
SQ = 256
D = 1024
HQ_LOCAL = 8
GQA = 4
DH = 128
SCALE = 0.08838834764831843
CHUNK = SQ // N_DEV
STEPS = N_DEV - 1


def _body(x_ref, wq_ref, wk_ref, wv_ref, wo_ref, out_ref,
          attn_ref, acc_ref, comm_ref, rs_send, rs_recv, ag_send, ag_recv):
    my = lax.axis_index("i")
    left = lax.rem(my + N_DEV - 1, N_DEV)
    right = lax.rem(my + 1, N_DEV)

    barrier = pltpu.get_barrier_semaphore()
    for nbr in (left, right):
        pl.semaphore_signal(barrier, inc=1, device_id=(nbr,),
                            device_id_type=pl.DeviceIdType.MESH)
    pl.semaphore_wait(barrier, 2)

    x = x_ref[...]
    q = jnp.dot(x, wq_ref[...], preferred_element_type=jnp.float32)
    k = jnp.dot(x, wk_ref[...], preferred_element_type=jnp.float32)
    v = jnp.dot(x, wv_ref[...], preferred_element_type=jnp.float32)
    for h in range(HQ_LOCAL):
        g = h // GQA
        qh = q[:, h * DH:(h + 1) * DH]
        kh = k[:, g * DH:(g + 1) * DH]
        vh = v[:, g * DH:(g + 1) * DH]
        s = jnp.dot(qh, kh.T, preferred_element_type=jnp.float32) * SCALE
        m = jnp.max(s, axis=-1, keepdims=True)
        p = jnp.exp(s - m)
        l = jnp.sum(p, axis=-1, keepdims=True)
        attn_ref[:, h * DH:(h + 1) * DH] = jnp.dot(
            p / l, vh, preferred_element_type=jnp.float32)

    acc_ref[...] = jnp.dot(attn_ref[...], wo_ref[...],
                           preferred_element_type=jnp.float32)

    for s in range(STEPS):
        c_send = lax.rem(my - s + 2 * N_DEV, N_DEV)
        rdma = pltpu.make_async_remote_copy(
            src_ref=acc_ref.at[pl.ds(c_send * CHUNK, CHUNK), :],
            dst_ref=comm_ref.at[s],
            send_sem=rs_send.at[s],
            recv_sem=rs_recv.at[s],
            device_id=(right,),
            device_id_type=pl.DeviceIdType.MESH,
        )
        rdma.start()
        rdma.wait()
        c_recv = lax.rem(my - s - 1 + 2 * N_DEV, N_DEV)
        rows = pl.ds(c_recv * CHUNK, CHUNK)
        acc_ref[rows, :] = acc_ref[rows, :] + comm_ref[s]

    own = lax.rem(my + 1, N_DEV)
    rows = pl.ds(own * CHUNK, CHUNK)
    out_ref[rows, :] = acc_ref[rows, :]

    for s in range(STEPS):
        c = lax.rem(my + 1 - s + 2 * N_DEV, N_DEV)
        sl = pl.ds(c * CHUNK, CHUNK)
        rdma = pltpu.make_async_remote_copy(
            src_ref=out_ref.at[sl, :],
            dst_ref=out_ref.at[sl, :],
            send_sem=ag_send.at[s],
            recv_sem=ag_recv.at[s],
            device_id=(right,),
            device_id_type=pl.DeviceIdType.MESH,
        )
        rdma.start()
        rdma.wait()


def kernel(x, Wq, Wo, Wk, Wv):
    i = lax.axis_index("i")
    x2 = x[0]
    wk_s = lax.dynamic_slice(Wk, (0, i * 2 * DH), (D, 2 * DH))
    wv_s = lax.dynamic_slice(Wv, (0, i * 2 * DH), (D, 2 * DH))
    out2 = pl.pallas_call(
        _body,
        out_shape=jax.ShapeDtypeStruct((SQ, D), jnp.float32),
        in_specs=[pl.BlockSpec(memory_space=pltpu.VMEM)] * 5,
        out_specs=pl.BlockSpec(memory_space=pltpu.VMEM),
        scratch_shapes=[
            pltpu.VMEM((SQ, HQ_LOCAL * DH), jnp.float32),
            pltpu.VMEM((SQ, D), jnp.float32),
            pltpu.VMEM((STEPS, CHUNK, D), jnp.float32),
            pltpu.SemaphoreType.DMA((STEPS,)),
            pltpu.SemaphoreType.DMA((STEPS,)),
            pltpu.SemaphoreType.DMA((STEPS,)),
            pltpu.SemaphoreType.DMA((STEPS,)),
        ],
        compiler_params=pltpu.CompilerParams(collective_id=0),
    )(x2, Wq, wk_s, wv_s, Wo)
    return out2[None]


# device time: 69053 ns/iter; 2.2123x vs baseline; 2.2123x over previous
import jax
import jax.numpy as jnp
from jax import lax
from jax.experimental import pallas as pl
from jax.experimental.pallas import tpu as pltpu

N_DEV = 32
SQ = 256
D = 1024
HQ_LOCAL = 8
GQA = 4
DH = 128
SCALE = 0.08838834764831843
CHUNK = SQ // N_DEV
STEPS = N_DEV - 1


RS_DISTS = (16, 8, 4, 2, 1)
AG_DISTS = (1, 2, 4, 8, 16)


def _body(x_ref, wq_ref, wk_ref, wv_ref, wo_ref, out_ref,
          attn_ref, comm_ref, rs_send, rs_recv, ag_send, ag_recv):
    my = lax.axis_index("i")

    barrier = pltpu.get_barrier_semaphore()
    for d in RS_DISTS:
        pl.semaphore_signal(barrier, inc=1, device_id=(my ^ d,),
                            device_id_type=pl.DeviceIdType.MESH)
    pl.semaphore_wait(barrier, len(RS_DISTS))

    x = x_ref[...]
    q = jnp.dot(x, wq_ref[...], preferred_element_type=jnp.float32)
    k = jnp.dot(x, wk_ref[...], preferred_element_type=jnp.float32)
    v = jnp.dot(x, wv_ref[...], preferred_element_type=jnp.float32)
    for h in range(HQ_LOCAL):
        g = h // GQA
        qh = q[:, h * DH:(h + 1) * DH]
        kh = k[:, g * DH:(g + 1) * DH]
        vh = v[:, g * DH:(g + 1) * DH]
        s = jnp.dot(qh, kh.T, preferred_element_type=jnp.float32) * SCALE
        m = jnp.max(s, axis=-1, keepdims=True)
        p = jnp.exp(s - m)
        l = jnp.sum(p, axis=-1, keepdims=True)
        attn_ref[:, h * DH:(h + 1) * DH] = jnp.dot(
            p / l, vh, preferred_element_type=jnp.float32)

    out_ref[...] = jnp.dot(attn_ref[...], wo_ref[...],
                           preferred_element_type=jnp.float32)

    base = my * 0
    seg = N_DEV
    off = 0
    for t, d in enumerate(RS_DISTS):
        half = seg // 2
        has = (my & d) != 0
        send_base = jnp.where(has, base, base + half)
        keep_base = jnp.where(has, base + half, base)
        rdma = pltpu.make_async_remote_copy(
            src_ref=out_ref.at[pl.ds(send_base * CHUNK, half * CHUNK), :],
            dst_ref=comm_ref.at[pl.ds(off * CHUNK, half * CHUNK), :],
            send_sem=rs_send.at[t],
            recv_sem=rs_recv.at[t],
            device_id=(my ^ d,),
            device_id_type=pl.DeviceIdType.MESH,
        )
        rdma.start()
        rdma.wait()
        rows = pl.ds(keep_base * CHUNK, half * CHUNK)
        out_ref[rows, :] = out_ref[rows, :] + \
            comm_ref[off * CHUNK:(off + half) * CHUNK, :]
        base = keep_base
        off += half
        seg = half

    for t, d in enumerate(AG_DISTS):
        seg = 1 << t
        sb = (my >> t) << t
        sl = pl.ds(sb * CHUNK, seg * CHUNK)
        rdma = pltpu.make_async_remote_copy(
            src_ref=out_ref.at[sl, :],
            dst_ref=out_ref.at[sl, :],
            send_sem=ag_send.at[t],
            recv_sem=ag_recv.at[t],
            device_id=(my ^ d,),
            device_id_type=pl.DeviceIdType.MESH,
        )
        rdma.start()
        rdma.wait()


def kernel(x, Wq, Wo, Wk, Wv):
    i = lax.axis_index("i")
    x2 = x[0]
    wk_s = lax.dynamic_slice(Wk, (0, i * 2 * DH), (D, 2 * DH))
    wv_s = lax.dynamic_slice(Wv, (0, i * 2 * DH), (D, 2 * DH))
    out2 = pl.pallas_call(
        _body,
        out_shape=jax.ShapeDtypeStruct((SQ, D), jnp.float32),
        in_specs=[pl.BlockSpec(memory_space=pltpu.VMEM)] * 5,
        out_specs=pl.BlockSpec(memory_space=pltpu.VMEM),
        scratch_shapes=[
            pltpu.VMEM((SQ, HQ_LOCAL * DH), jnp.float32),
            pltpu.VMEM((STEPS * CHUNK, D), jnp.float32),
            pltpu.SemaphoreType.DMA((len(RS_DISTS),)),
            pltpu.SemaphoreType.DMA((len(RS_DISTS),)),
            pltpu.SemaphoreType.DMA((len(AG_DISTS),)),
            pltpu.SemaphoreType.DMA((len(AG_DISTS),)),
        ],
        compiler_params=pltpu.CompilerParams(collective_id=0),
    )(x2, Wq, wk_s, wv_s, Wo)
    return out2[None]
